# Initial kernel scaffold; baseline (speedup 1.0000x reference)
import jax
import jax.numpy as jnp
from jax.experimental import pallas as pl


def kernel(x, edge_index, W1, b1, W2, b2, W3, b3):
    raise NotImplementedError("write your pallas kernel here")


# speedup vs baseline: 11.6216x; 11.6216x over previous
import functools

import jax
import jax.numpy as jnp
from jax import lax
from jax.experimental import pallas as pl
from jax.experimental.pallas import tpu as pltpu
from jax.experimental.pallas import tpu_sc as plsc

N = 10000
D = 128
NC = 2
NS = 16
NW = NC * NS
CHUNK = 128
NPAD = 10240
STRIPE = NPAD // NS
DUMP = N


_HROWS = NPAD // 128
_HSTRIPE = _HROWS // NS


def _deg_body(di_hbm, zer_hbm, out_hbm, di_v, h1_v, h2_v, idx_v, acc_s):
    c = lax.axis_index("c")
    s = lax.axis_index("s")
    w = c * NS + s
    nchunks = di_hbm.shape[1]
    pltpu.sync_copy(di_hbm.at[w], di_v)

    zv = jnp.zeros((16,), jnp.float32)

    def zfill(i, _):
        h1_v[pl.ds(i * 16, 16)] = zv
        return 0

    lax.fori_loop(0, NPAD // 16, zfill, 0)

    def ifill(k, _):
        idx_v[pl.ds(k * 16, 16)] = lax.iota(jnp.int32, 16) + k * 16
        return 0

    lax.fori_loop(0, _HROWS // 16, ifill, 0)

    @pl.when(s < _HROWS // 8)
    def _():
        pltpu.sync_copy(zer_hbm.at[pl.ds(0, 8)],
                        acc_s.at[pl.ds(s * 8, 8)])

    ones = jnp.full((16,), 1.0, jnp.float32)

    def hadd(i, _):
        j = i // 8
        k = i - j * 8
        dv = di_v[j, pl.ds(k * 16, 16)]
        plsc.addupdate_scatter(h1_v, [dv], ones)
        return 0

    lax.fori_loop(0, nchunks * 8, hadd, 0)

    def rsh(i, _):
        r = i // 8
        k = i - r * 8
        h2_v[r, pl.ds(k * 16, 16)] = h1_v[pl.ds(i * 16, 16)]
        return 0

    lax.fori_loop(0, NPAD // 16, rsh, 0)
    plsc.subcore_barrier()
    pltpu.sync_copy(h2_v, acc_s.at[idx_v], add=True)
    plsc.subcore_barrier()

    @pl.when(s < _HROWS // 8)
    def _():
        pltpu.sync_copy(acc_s.at[pl.ds(s * 8, 8)],
                        out_hbm.at[c].at[pl.ds(s * 8, 8)])


def _sc_degree(di3, zer):
    nchunks = di3.shape[1]
    mesh = plsc.VectorSubcoreMesh(core_axis_name="c", subcore_axis_name="s")
    f = pl.kernel(
        _deg_body,
        out_type=jax.ShapeDtypeStruct((NC, _HROWS, 128), jnp.float32),
        mesh=mesh,
        scratch_types=[
            pltpu.VMEM((nchunks, CHUNK), jnp.int32),
            pltpu.VMEM((NPAD,), jnp.float32),
            pltpu.VMEM((_HROWS, 128), jnp.float32),
            pltpu.VMEM((_HROWS,), jnp.int32),
            pltpu.VMEM_SHARED((_HROWS, 128), jnp.float32),
        ],
        compiler_params=pltpu.CompilerParams(needs_layout_passes=False),
    )
    return f(di3, zer)


def _agg_body(h_hbm, si_hbm, di_hbm, zer_hbm, out_hbm,
              si_v, di_v, rows_v, acc_s, sem):
    c = lax.axis_index("c")
    s = lax.axis_index("s")
    w = c * NS + s
    nchunks = si_hbm.shape[1]
    pltpu.sync_copy(si_hbm.at[w], si_v)
    pltpu.sync_copy(di_hbm.at[w], di_v)

    @pl.when(c == 0)
    def _():
        pltpu.sync_copy(h_hbm.at[pl.ds(s * STRIPE, STRIPE)],
                        acc_s.at[pl.ds(s * STRIPE, STRIPE)])

    @pl.when(c != 0)
    def _():
        pltpu.sync_copy(zer_hbm, acc_s.at[pl.ds(s * STRIPE, STRIPE)])

    plsc.subcore_barrier()

    def body(j, _):
        pltpu.async_copy(h_hbm.at[si_v.at[j]], rows_v, sem).wait()
        pltpu.sync_copy(rows_v, acc_s.at[di_v.at[j]], add=True)
        return 0

    lax.fori_loop(0, nchunks, body, 0)
    plsc.subcore_barrier()
    pltpu.sync_copy(acc_s.at[pl.ds(s * STRIPE, STRIPE)],
                    out_hbm.at[c].at[pl.ds(s * STRIPE, STRIPE)])


def _sc_aggregate(h, si3, di3, zer):
    nchunks = si3.shape[1]
    mesh = plsc.VectorSubcoreMesh(core_axis_name="c", subcore_axis_name="s")
    f = pl.kernel(
        _agg_body,
        out_type=jax.ShapeDtypeStruct((NC, NPAD, D), jnp.float32),
        mesh=mesh,
        scratch_types=[
            pltpu.VMEM((nchunks, CHUNK), jnp.int32),
            pltpu.VMEM((nchunks, CHUNK), jnp.int32),
            pltpu.VMEM((CHUNK, D), jnp.float32),
            pltpu.VMEM_SHARED((NPAD, D), jnp.float32),
            pltpu.SemaphoreType.DMA,
        ],
    )
    return f(h, si3, di3, zer)


_BLK = 1024
_GRID = NPAD // _BLK


def _mm1_body(d0_ref, d1_ref, x_ref, w_ref, o_ref, dis_ref):
    deg = 1.0 + d0_ref[...] + d1_ref[...]
    dis = lax.rsqrt(deg)
    dis_ref[...] = dis
    o_ref[...] = jnp.dot(x_ref[...], w_ref[...],
                         preferred_element_type=jnp.float32) * dis


def _tc_first(d0, d1, x, w):
    return pl.pallas_call(
        _mm1_body,
        grid=(_GRID,),
        in_specs=[
            pl.BlockSpec((_BLK, 1), lambda i: (i, 0)),
            pl.BlockSpec((_BLK, 1), lambda i: (i, 0)),
            pl.BlockSpec((_BLK, D), lambda i: (i, 0)),
            pl.BlockSpec((D, D), lambda i: (0, 0)),
        ],
        out_specs=[
            pl.BlockSpec((_BLK, D), lambda i: (i, 0)),
            pl.BlockSpec((_BLK, 1), lambda i: (i, 0)),
        ],
        out_shape=[
            jax.ShapeDtypeStruct((NPAD, D), jnp.float32),
            jax.ShapeDtypeStruct((NPAD, 1), jnp.float32),
        ],
    )(d0, d1, x, w)


def _layer_body(p0_ref, p1_ref, dis_ref, b_ref, w_ref, o_ref):
    dis = dis_ref[...]
    t = (p0_ref[...] + p1_ref[...]) * dis + b_ref[...]
    a = jnp.maximum(t, 0.0)
    o_ref[...] = jnp.dot(a, w_ref[...],
                         preferred_element_type=jnp.float32) * dis


def _tc_layer(p, dis, b, w):
    return pl.pallas_call(
        _layer_body,
        grid=(_GRID,),
        in_specs=[
            pl.BlockSpec((_BLK, D), lambda i: (i, 0)),
            pl.BlockSpec((_BLK, D), lambda i: (i, 0)),
            pl.BlockSpec((_BLK, 1), lambda i: (i, 0)),
            pl.BlockSpec((1, D), lambda i: (0, 0)),
            pl.BlockSpec((D, D), lambda i: (0, 0)),
        ],
        out_specs=pl.BlockSpec((_BLK, D), lambda i: (i, 0)),
        out_shape=jax.ShapeDtypeStruct((NPAD, D), jnp.float32),
    )(p[0], p[1], dis, b, w)


def _final_body(p0_ref, p1_ref, dis_ref, b_ref, o_ref):
    t = (p0_ref[...] + p1_ref[...]) * dis_ref[...] + b_ref[...]
    m = jnp.max(t, axis=1, keepdims=True)
    e = jnp.exp(t - m)
    ssum = jnp.sum(e, axis=1, keepdims=True)
    o_ref[...] = t - m - jnp.log(ssum)


def _tc_final(p, dis, b):
    return pl.pallas_call(
        _final_body,
        grid=(_GRID,),
        in_specs=[
            pl.BlockSpec((_BLK, D), lambda i: (i, 0)),
            pl.BlockSpec((_BLK, D), lambda i: (i, 0)),
            pl.BlockSpec((_BLK, 1), lambda i: (i, 0)),
            pl.BlockSpec((1, D), lambda i: (0, 0)),
        ],
        out_specs=pl.BlockSpec((_BLK, D), lambda i: (i, 0)),
        out_shape=jax.ShapeDtypeStruct((NPAD, D), jnp.float32),
    )(p[0], p[1], dis, b)


@jax.jit
def kernel(x, edge_index, W1, b1, W2, b2, W3, b3):
    E = edge_index.shape[1]
    nchunks = -(-E // (NW * CHUNK))
    epad = NW * nchunks * CHUNK

    src = edge_index[0].astype(jnp.int32)
    dst = edge_index[1].astype(jnp.int32)
    si3 = jnp.pad(src, (0, epad - E)).reshape(NW, nchunks, CHUNK)
    di3 = jnp.pad(dst, (0, epad - E),
                  constant_values=DUMP).reshape(NW, nchunks, CHUNK)

    x_p = jnp.pad(x, ((0, NPAD - N), (0, 0)))
    zer = jnp.zeros((STRIPE, D), jnp.float32)
    b1r = b1.reshape(1, D)
    b2r = b2.reshape(1, D)
    b3r = b3.reshape(1, D)

    degp = _sc_degree(di3, zer)
    tmp1, dis = _tc_first(degp[0].reshape(NPAD, 1), degp[1].reshape(NPAD, 1),
                          x_p, W1)
    p1 = _sc_aggregate(tmp1, si3, di3, zer)
    tmp2 = _tc_layer(p1, dis, b1r, W2)
    p2 = _sc_aggregate(tmp2, si3, di3, zer)
    tmp3 = _tc_layer(p2, dis, b2r, W3)
    p3 = _sc_aggregate(tmp3, si3, di3, zer)
    out = _tc_final(p3, dis, b3r)
    return out[:N]
